# baseline (speedup 1.0000x reference)
import dataclasses
import functools

import jax
import jax.numpy as jnp
from jax import lax
from jax.experimental import pallas as pl
from jax.experimental.pallas import tpu as pltpu
from jax.experimental.pallas import tpu_sc as plsc

K = 5
LANES = 16
NT = 16
MINF = float("-inf")
RBITS = 14


def _tc_body(x_ref, wt_ref, ad_ref, as_ref, xp_ref, a_ref, b_ref):
    xp = jnp.dot(x_ref[...], wt_ref[...], preferred_element_type=jnp.float32)
    xp_ref[...] = xp
    a_ref[...] = jnp.dot(xp, ad_ref[...], preferred_element_type=jnp.float32,
                         precision=jax.lax.Precision.HIGHEST)
    b_ref[...] = jnp.dot(xp, as_ref[...], preferred_element_type=jnp.float32,
                         precision=jax.lax.Precision.HIGHEST)


@functools.lru_cache(maxsize=None)
def _make_tc(N, IN, C):
    BLK = 2000
    assert N % BLK == 0
    grid = (N // BLK,)
    return pl.pallas_call(
        _tc_body,
        grid=grid,
        in_specs=[
            pl.BlockSpec((BLK, IN), lambda i: (i, 0)),
            pl.BlockSpec((IN, C), lambda i: (0, 0)),
            pl.BlockSpec((C, 1), lambda i: (0, 0)),
            pl.BlockSpec((C, 1), lambda i: (0, 0)),
        ],
        out_specs=[
            pl.BlockSpec((BLK, C), lambda i: (i, 0)),
            pl.BlockSpec((BLK, 1), lambda i: (i, 0)),
            pl.BlockSpec((BLK, 1), lambda i: (i, 0)),
        ],
        out_shape=[
            jax.ShapeDtypeStruct((N, C), jnp.float32),
            jax.ShapeDtypeStruct((N, 1), jnp.float32),
            jax.ShapeDtypeStruct((N, 1), jnp.float32),
        ],
    )


@functools.lru_cache(maxsize=None)
def _make_sc(N, C, EPAD, NPAD):
    EPT = EPAD // NT
    NPT = NPAD // NT
    NV_E = EPT // LANES
    NV_N = NPT // LANES
    NV_P = NPAD // LANES
    CH = 16
    NCH = NPT // CH
    CL = C // LANES
    RMASK = (1 << RBITS) - 1

    mesh = plsc.VectorSubcoreMesh(
        core_axis_name="c", subcore_axis_name="s", num_cores=1,
        num_subcores=NT)

    def body(pe_hbm, a_hbm, b_hbm, xp_hbm, out_hbm,
             gm, mk, ck, colk, wk, selcol, sh16f, sh16i, spub, sem):
        tid = lax.axis_index("s")

        def fill(ref, nvec, val, dtype):
            def fb(i, _):
                ref[pl.ds(i * LANES, LANES)] = jnp.full((LANES,), val, dtype)
                return 0
            lax.fori_loop(0, nvec, fb, 0)

        def selection(pev, bnode):
            with jax.named_scope("prologue"):
                pltpu.sync_copy(pe_hbm.at[pl.ds(tid * EPT, EPT)], pev)
                pltpu.sync_copy(b_hbm, bnode)

            def rmw_max(rows, v, macc):
                g = plsc.load_gather(macc, [rows])

                def cond(gc):
                    return jnp.any(v > gc)

                def bdy(gc):
                    plsc.store_scatter(macc, [rows], jnp.maximum(gc, v),
                                       mask=v > gc)
                    return plsc.load_gather(macc, [rows])

                lax.while_loop(cond, bdy, g)

            def max_pass0(macc):
                def eb(i, _):
                    st = i * LANES
                    pe = pev[pl.ds(st, LANES)]
                    rows = pe & RMASK
                    cols = lax.shift_right_logical(pe, RBITS)
                    be = plsc.load_gather(bnode, [cols])
                    rmw_max(rows, be, macc)
                    return 0
                lax.fori_loop(0, NV_E, eb, 0)

            def fused_pass(cnt, colc, macc, with_max):
                ones = jnp.ones((LANES,), jnp.int32)

                def eb(i, _):
                    st = i * LANES
                    pe = pev[pl.ds(st, LANES)]
                    rows = pe & RMASK
                    cols = lax.shift_right_logical(pe, RBITS)
                    be = plsc.load_gather(bnode, [cols])
                    mv = plsc.load_gather(gm, [rows])
                    eq = be == mv
                    plsc.addupdate_scatter(cnt, [rows], ones, mask=eq)
                    plsc.store_scatter(colc, [rows], cols, mask=eq)
                    if with_max:
                        v = jnp.where(
                            be < mv, be,
                            jnp.full((LANES,), MINF, jnp.float32))
                        rmw_max(rows, v, macc)
                    return 0
                lax.fori_loop(0, NV_E, eb, 0)

            def merge_max(k, macc):
                stagef = macc
                pltpu.sync_copy(macc, sh16f.at[pl.ds(tid * NPAD, NPAD)])
                plsc.subcore_barrier()
                for j in range(NT):
                    pltpu.sync_copy(
                        sh16f.at[pl.ds(j * NPAD + tid * NPT, NPT)],
                        stagef.at[pl.ds(j * NPT, NPT)])

                def rb(v, _):
                    st = v * LANES
                    red = stagef[pl.ds(st, LANES)]
                    for j in range(1, NT):
                        red = jnp.maximum(red,
                                          stagef[pl.ds(j * NPT + st, LANES)])
                    mk[pl.ds(k * NPT + st, LANES)] = red
                    return 0
                lax.fori_loop(0, NV_N, rb, 0)
                plsc.subcore_barrier()
                pltpu.sync_copy(mk.at[pl.ds(k * NPT, NPT)],
                                spub.at[pl.ds(tid * NPT, NPT)])
                plsc.subcore_barrier()
                pltpu.sync_copy(spub, gm)
                plsc.subcore_barrier()

            def merge_cnts(k, cnt, colc):
                stagei = cnt
                pltpu.sync_copy(cnt, sh16i.at[pl.ds(tid * 2 * NPAD, NPAD)])
                pltpu.sync_copy(colc,
                                sh16i.at[pl.ds(tid * 2 * NPAD + NPAD, NPAD)])
                plsc.subcore_barrier()
                for j in range(NT):
                    pltpu.sync_copy(
                        sh16i.at[pl.ds(2 * j * NPAD + tid * NPT, NPT)],
                        stagei.at[pl.ds(j * NPT, NPT)])

                def rbc(v, _):
                    st = v * LANES
                    redc = stagei[pl.ds(st, LANES)]
                    for j in range(1, NT):
                        redc = redc + stagei[pl.ds(j * NPT + st, LANES)]
                    ck[pl.ds(k * NPT + st, LANES)] = redc
                    return 0
                lax.fori_loop(0, NV_N, rbc, 0)
                for j in range(NT):
                    pltpu.sync_copy(
                        sh16i.at[pl.ds((2 * j + 1) * NPAD + tid * NPT, NPT)],
                        stagei.at[pl.ds(j * NPT, NPT)])

                def rbl(v, _):
                    st = v * LANES
                    redl = stagei[pl.ds(st, LANES)]
                    for j in range(1, NT):
                        redl = jnp.maximum(redl,
                                           stagei[pl.ds(j * NPT + st, LANES)])
                    colk[pl.ds(k * NPT + st, LANES)] = redl
                    return 0
                lax.fori_loop(0, NV_N, rbl, 0)
                plsc.subcore_barrier()

            def rounds(cnt, colc, macc):
                with jax.named_scope("p0_fill"):
                    fill(macc, NV_P, MINF, jnp.float32)
                with jax.named_scope("p0_max"):
                    max_pass0(macc)
                with jax.named_scope("p0_merge"):
                    merge_max(0, macc)
                for k in range(1, K + 1):
                    with_max = k < K
                    with jax.named_scope(f"r{k}_fill"):
                        fill(cnt, NV_P, 0, jnp.int32)
                        fill(colc, NV_P, -1, jnp.int32)
                        if with_max:
                            fill(macc, NV_P, MINF, jnp.float32)
                    with jax.named_scope(f"r{k}_pass"):
                        fused_pass(cnt, colc, macc, with_max)
                    with jax.named_scope(f"r{k}_mcnt"):
                        merge_cnts(k - 1, cnt, colc)
                    if with_max:
                        with jax.named_scope(f"r{k}_mmax"):
                            merge_max(k, macc)
            pl.run_scoped(rounds,
                          pltpu.VMEM((NPAD,), jnp.int32),
                          pltpu.VMEM((NPAD,), jnp.int32),
                          pltpu.VMEM((NPAD,), jnp.float32))

        pl.run_scoped(selection,
                      pltpu.VMEM((EPT,), jnp.int32),
                      pltpu.VMEM((NPAD,), jnp.float32))

        def weights(anode):
            pltpu.sync_copy(a_hbm.at[pl.ds(tid * NPT, NPT)], anode)

            def wb(v, _):
                st = v * LANES
                a16 = anode[pl.ds(st, LANES)]
                rem = jnp.full((LANES,), K, jnp.int32)
                denom = jnp.zeros((LANES,), jnp.float32)
                exs = []
                s0 = None
                for k in range(K):
                    m = mk[pl.ds(k * NPT + st, LANES)]
                    c = ck[pl.ds(k * NPT + st, LANES)]
                    s = a16 + m
                    s = jnp.where(s >= 0, s, 0.2 * s)
                    if k == 0:
                        s0 = s
                    t = jnp.minimum(c, jnp.maximum(rem, 0))
                    rem = rem - t
                    valid = t > 0
                    ex = jnp.where(valid,
                                   t.astype(jnp.float32) * jnp.exp(s - s0),
                                   jnp.zeros((LANES,), jnp.float32))
                    denom = denom + ex
                    exs.append(ex)
                    cv = colk[pl.ds(k * NPT + st, LANES)]
                    selcol[pl.ds(k * NPT + st, LANES)] = jnp.where(
                        cv >= 0, cv, jnp.zeros((LANES,), jnp.int32))
                for k in range(K):
                    wk[pl.ds(k * NPT + st, LANES)] = exs[k] / denom
                return 0
            lax.fori_loop(0, NV_N, wb, 0)
        with jax.named_scope("weights"):
            pl.run_scoped(weights, pltpu.VMEM((NPT,), jnp.float32))

        def aggregate(rowbuf, outchunk):
            def fire(ch, buf):
                descs = []
                for k in range(K):
                    d = pltpu.async_copy(
                        xp_hbm.at[selcol.at[pl.ds(k * NPT + ch * CH, CH)]],
                        rowbuf.at[pl.ds((buf * K + k) * CH, CH)], sem)
                    descs.append(d)
                return descs

            for ch in range(NCH):
                buf = 0
                for d in fire(ch, buf):
                    d.wait()

                def nbody(n, _):
                    zero = jnp.zeros((LANES,), jnp.float32)
                    accs0 = tuple(zero for _ in range(CL))

                    def kbody(k, accs):
                        wv = wk[pl.ds(k * NPT + ch * CH + n, LANES)]
                        wb16 = jnp.full((LANES,), wv[0])
                        out = []
                        for c in range(CL):
                            r = rowbuf[(buf * K + k) * CH + n,
                                       pl.ds(c * LANES, LANES)]
                            out.append(accs[c] + wb16 * r)
                        return tuple(out)

                    accs = lax.fori_loop(0, K, kbody, accs0)
                    for c in range(CL):
                        v = accs[c]
                        ev = jnp.where(
                            v > 0, v,
                            jnp.exp(jnp.minimum(v, 0.0)) - 1.0)
                        outchunk[n, pl.ds(c * LANES, LANES)] = ev
                    return 0
                lax.fori_loop(0, CH, nbody, 0)
                pltpu.sync_copy(outchunk,
                                out_hbm.at[pl.ds(tid * NPT + ch * CH, CH), :])
        with jax.named_scope("agg"):
            pl.run_scoped(aggregate,
                      pltpu.VMEM((K * CH, C), jnp.float32),
                      pltpu.VMEM((CH, C), jnp.float32))

    cp = pltpu.CompilerParams()
    if "needs_layout_passes" in pltpu.CompilerParams.__dataclass_fields__:
        cp = dataclasses.replace(cp, needs_layout_passes=False)

    return pl.kernel(
        body,
        out_type=jax.ShapeDtypeStruct((NPAD, C), jnp.float32),
        mesh=mesh,
        compiler_params=cp,
        scratch_types=[
            pltpu.VMEM((NPAD,), jnp.float32),
            pltpu.VMEM((K * NPT,), jnp.float32),
            pltpu.VMEM((K * NPT,), jnp.int32),
            pltpu.VMEM((K * NPT,), jnp.int32),
            pltpu.VMEM((K * NPT + LANES,), jnp.float32),
            pltpu.VMEM((K * NPT,), jnp.int32),
            pltpu.VMEM_SHARED((NT * NPAD,), jnp.float32),
            pltpu.VMEM_SHARED((2 * NT * NPAD,), jnp.int32),
            pltpu.VMEM_SHARED((NPAD,), jnp.float32),
            pltpu.SemaphoreType.DMA,
        ],
    )


def kernel(x, edge_index, W, att):
    N, IN = x.shape
    C = W.shape[0]
    E = edge_index.shape[1]

    NPAD = ((N + NT * LANES - 1) // (NT * LANES)) * NT * LANES
    EP = E + N
    EPT = ((EP + NT * LANES - 1) // (NT * LANES)) * LANES
    EPAD = EPT * NT

    WT = W.T
    attd = att[0, 0, :C].reshape(C, 1)
    atts = att[0, 0, C:].reshape(C, 1)

    xp, a1, b1 = _make_tc(N, IN, C)(x, WT, attd, atts)
    a = jnp.pad(a1[:, 0], (0, NPAD - N))
    b = jnp.pad(b1[:, 0], (0, NPAD - N))

    loops = jnp.arange(N, dtype=jnp.int32)
    rowp = jnp.concatenate(
        [edge_index[0], loops,
         jnp.full((EPAD - EP,), N, jnp.int32)])
    colp = jnp.concatenate(
        [edge_index[1], loops,
         jnp.zeros((EPAD - EP,), jnp.int32)])
    packed = rowp | (colp << RBITS)

    out = _make_sc(N, C, EPAD, NPAD)(packed, a, b, xp)
    return out[:N]


# speedup vs baseline: 1.1896x; 1.1896x over previous
import dataclasses
import functools

import jax
import jax.numpy as jnp
from jax import lax
from jax.experimental import pallas as pl
from jax.experimental.pallas import tpu as pltpu
from jax.experimental.pallas import tpu_sc as plsc

K = 5
LANES = 16
NT = 16
MINF = float("-inf")
RBITS = 14


def _tc_body(x_ref, wt_ref, ad_ref, as_ref, xp_ref, a_ref, b_ref):
    xp = jnp.dot(x_ref[...], wt_ref[...], preferred_element_type=jnp.float32)
    xp_ref[...] = xp
    a_ref[...] = jnp.dot(xp, ad_ref[...], preferred_element_type=jnp.float32,
                         precision=jax.lax.Precision.HIGHEST)
    b_ref[...] = jnp.dot(xp, as_ref[...], preferred_element_type=jnp.float32,
                         precision=jax.lax.Precision.HIGHEST)


@functools.lru_cache(maxsize=None)
def _make_tc(N, IN, C):
    BLK = 2000
    assert N % BLK == 0
    grid = (N // BLK,)
    return pl.pallas_call(
        _tc_body,
        grid=grid,
        in_specs=[
            pl.BlockSpec((BLK, IN), lambda i: (i, 0)),
            pl.BlockSpec((IN, C), lambda i: (0, 0)),
            pl.BlockSpec((C, 1), lambda i: (0, 0)),
            pl.BlockSpec((C, 1), lambda i: (0, 0)),
        ],
        out_specs=[
            pl.BlockSpec((BLK, C), lambda i: (i, 0)),
            pl.BlockSpec((BLK, 1), lambda i: (i, 0)),
            pl.BlockSpec((BLK, 1), lambda i: (i, 0)),
        ],
        out_shape=[
            jax.ShapeDtypeStruct((N, C), jnp.float32),
            jax.ShapeDtypeStruct((N, 1), jnp.float32),
            jax.ShapeDtypeStruct((N, 1), jnp.float32),
        ],
    )


@functools.lru_cache(maxsize=None)
def _make_sc(N, C, EPAD, NPAD):
    EPT = EPAD // NT
    NPT = NPAD // NT
    NV_E = EPT // LANES
    NV_N = NPT // LANES
    NV_P = NPAD // LANES
    CH = 16
    NCH = NPT // CH
    CL = C // LANES
    RMASK = (1 << RBITS) - 1

    mesh = plsc.VectorSubcoreMesh(
        core_axis_name="c", subcore_axis_name="s", num_cores=1,
        num_subcores=NT)

    def body(pe_hbm, a_hbm, b_hbm, xp_hbm, out_hbm,
             gm, mk, ck, colk, wk, selcol, sh16f, sh16i, spub, sem):
        tid = lax.axis_index("s")

        def fill(ref, nvec, val, dtype):
            def fb(i, _):
                ref[pl.ds(i * LANES, LANES)] = jnp.full((LANES,), val, dtype)
                return 0
            lax.fori_loop(0, nvec, fb, 0)

        def selection(pev, bnode):
            with jax.named_scope("prologue"):
                pltpu.sync_copy(pe_hbm.at[pl.ds(tid * EPT, EPT)], pev)
                pltpu.sync_copy(b_hbm, bnode)

            def rmw_max(rows, v, macc):
                g = plsc.load_gather(macc, [rows])
                plsc.store_scatter(macc, [rows], jnp.maximum(g, v),
                                   mask=v > g)
                g2 = plsc.load_gather(macc, [rows])

                @pl.when(jnp.any(v > g2))
                def _slow():
                    def cond(gc):
                        return jnp.any(v > gc)

                    def bdy(gc):
                        plsc.store_scatter(macc, [rows], jnp.maximum(gc, v),
                                           mask=v > gc)
                        return plsc.load_gather(macc, [rows])

                    lax.while_loop(cond, bdy, g2)

            def max_pass0(macc):
                def eb(i, _):
                    st = i * LANES
                    pe = pev[pl.ds(st, LANES)]
                    rows = pe & RMASK
                    cols = lax.shift_right_logical(pe, RBITS)
                    be = plsc.load_gather(bnode, [cols])
                    rmw_max(rows, be, macc)
                    return 0
                lax.fori_loop(0, NV_E, eb, 0)

            def fused_pass(cnt, colc, macc, with_max):
                ones = jnp.ones((LANES,), jnp.int32)

                def eb(i, _):
                    st = i * LANES
                    pe = pev[pl.ds(st, LANES)]
                    rows = pe & RMASK
                    cols = lax.shift_right_logical(pe, RBITS)
                    be = plsc.load_gather(bnode, [cols])
                    mv = plsc.load_gather(gm, [rows])
                    eq = be == mv
                    plsc.addupdate_scatter(cnt, [rows], ones, mask=eq)
                    plsc.store_scatter(colc, [rows], cols, mask=eq)
                    if with_max:
                        v = jnp.where(
                            be < mv, be,
                            jnp.full((LANES,), MINF, jnp.float32))
                        rmw_max(rows, v, macc)
                    return 0
                lax.fori_loop(0, NV_E, eb, 0)

            def merge_max(k, macc):
                stagef = macc
                pltpu.sync_copy(macc, sh16f.at[pl.ds(tid * NPAD, NPAD)])
                plsc.subcore_barrier()
                for j in range(NT):
                    pltpu.sync_copy(
                        sh16f.at[pl.ds(j * NPAD + tid * NPT, NPT)],
                        stagef.at[pl.ds(j * NPT, NPT)])

                def rb(v, _):
                    st = v * LANES
                    red = stagef[pl.ds(st, LANES)]
                    for j in range(1, NT):
                        red = jnp.maximum(red,
                                          stagef[pl.ds(j * NPT + st, LANES)])
                    mk[pl.ds(k * NPT + st, LANES)] = red
                    return 0
                lax.fori_loop(0, NV_N, rb, 0)
                plsc.subcore_barrier()
                pltpu.sync_copy(mk.at[pl.ds(k * NPT, NPT)],
                                spub.at[pl.ds(tid * NPT, NPT)])
                plsc.subcore_barrier()
                pltpu.sync_copy(spub, gm)
                plsc.subcore_barrier()

            def merge_cnts(k, cnt, colc):
                stagei = cnt
                pltpu.sync_copy(cnt, sh16i.at[pl.ds(tid * 2 * NPAD, NPAD)])
                pltpu.sync_copy(colc,
                                sh16i.at[pl.ds(tid * 2 * NPAD + NPAD, NPAD)])
                plsc.subcore_barrier()
                for j in range(NT):
                    pltpu.sync_copy(
                        sh16i.at[pl.ds(2 * j * NPAD + tid * NPT, NPT)],
                        stagei.at[pl.ds(j * NPT, NPT)])

                def rbc(v, _):
                    st = v * LANES
                    redc = stagei[pl.ds(st, LANES)]
                    for j in range(1, NT):
                        redc = redc + stagei[pl.ds(j * NPT + st, LANES)]
                    ck[pl.ds(k * NPT + st, LANES)] = redc
                    return 0
                lax.fori_loop(0, NV_N, rbc, 0)
                for j in range(NT):
                    pltpu.sync_copy(
                        sh16i.at[pl.ds((2 * j + 1) * NPAD + tid * NPT, NPT)],
                        stagei.at[pl.ds(j * NPT, NPT)])

                def rbl(v, _):
                    st = v * LANES
                    redl = stagei[pl.ds(st, LANES)]
                    for j in range(1, NT):
                        redl = jnp.maximum(redl,
                                           stagei[pl.ds(j * NPT + st, LANES)])
                    colk[pl.ds(k * NPT + st, LANES)] = redl
                    return 0
                lax.fori_loop(0, NV_N, rbl, 0)
                plsc.subcore_barrier()

            def rounds(cnt, colc, macc):
                with jax.named_scope("p0_fill"):
                    fill(macc, NV_P, MINF, jnp.float32)
                with jax.named_scope("p0_max"):
                    max_pass0(macc)
                with jax.named_scope("p0_merge"):
                    merge_max(0, macc)
                for k in range(1, K + 1):
                    with_max = k < K
                    with jax.named_scope(f"r{k}_fill"):
                        fill(cnt, NV_P, 0, jnp.int32)
                        fill(colc, NV_P, -1, jnp.int32)
                        if with_max:
                            fill(macc, NV_P, MINF, jnp.float32)
                    with jax.named_scope(f"r{k}_pass"):
                        fused_pass(cnt, colc, macc, with_max)
                    with jax.named_scope(f"r{k}_mcnt"):
                        merge_cnts(k - 1, cnt, colc)
                    if with_max:
                        with jax.named_scope(f"r{k}_mmax"):
                            merge_max(k, macc)
            pl.run_scoped(rounds,
                          pltpu.VMEM((NPAD,), jnp.int32),
                          pltpu.VMEM((NPAD,), jnp.int32),
                          pltpu.VMEM((NPAD,), jnp.float32))

        pl.run_scoped(selection,
                      pltpu.VMEM((EPT,), jnp.int32),
                      pltpu.VMEM((NPAD,), jnp.float32))

        def weights(anode):
            pltpu.sync_copy(a_hbm.at[pl.ds(tid * NPT, NPT)], anode)

            def wb(v, _):
                st = v * LANES
                a16 = anode[pl.ds(st, LANES)]
                rem = jnp.full((LANES,), K, jnp.int32)
                denom = jnp.zeros((LANES,), jnp.float32)
                exs = []
                s0 = None
                for k in range(K):
                    m = mk[pl.ds(k * NPT + st, LANES)]
                    c = ck[pl.ds(k * NPT + st, LANES)]
                    s = a16 + m
                    s = jnp.where(s >= 0, s, 0.2 * s)
                    if k == 0:
                        s0 = s
                    t = jnp.minimum(c, jnp.maximum(rem, 0))
                    rem = rem - t
                    valid = t > 0
                    ex = jnp.where(valid,
                                   t.astype(jnp.float32) * jnp.exp(s - s0),
                                   jnp.zeros((LANES,), jnp.float32))
                    denom = denom + ex
                    exs.append(ex)
                    cv = colk[pl.ds(k * NPT + st, LANES)]
                    selcol[pl.ds(k * NPT + st, LANES)] = jnp.where(
                        cv >= 0, cv, jnp.zeros((LANES,), jnp.int32))
                for k in range(K):
                    wk[pl.ds(k * NPT + st, LANES)] = exs[k] / denom
                return 0
            lax.fori_loop(0, NV_N, wb, 0)
        with jax.named_scope("weights"):
            pl.run_scoped(weights, pltpu.VMEM((NPT,), jnp.float32))

        def aggregate(rowbuf, outchunk):
            def fire(ch, buf):
                descs = []
                for k in range(K):
                    d = pltpu.async_copy(
                        xp_hbm.at[selcol.at[pl.ds(k * NPT + ch * CH, CH)]],
                        rowbuf.at[pl.ds((buf * K + k) * CH, CH)], sem)
                    descs.append(d)
                return descs

            for ch in range(NCH):
                buf = 0
                for d in fire(ch, buf):
                    d.wait()

                def nbody(n, _):
                    zero = jnp.zeros((LANES,), jnp.float32)
                    accs0 = tuple(zero for _ in range(CL))

                    def kbody(k, accs):
                        wv = wk[pl.ds(k * NPT + ch * CH + n, LANES)]
                        wb16 = jnp.full((LANES,), wv[0])
                        out = []
                        for c in range(CL):
                            r = rowbuf[(buf * K + k) * CH + n,
                                       pl.ds(c * LANES, LANES)]
                            out.append(accs[c] + wb16 * r)
                        return tuple(out)

                    accs = lax.fori_loop(0, K, kbody, accs0)
                    for c in range(CL):
                        v = accs[c]
                        ev = jnp.where(
                            v > 0, v,
                            jnp.exp(jnp.minimum(v, 0.0)) - 1.0)
                        outchunk[n, pl.ds(c * LANES, LANES)] = ev
                    return 0
                lax.fori_loop(0, CH, nbody, 0)
                pltpu.sync_copy(outchunk,
                                out_hbm.at[pl.ds(tid * NPT + ch * CH, CH), :])
        with jax.named_scope("agg"):
            pl.run_scoped(aggregate,
                      pltpu.VMEM((K * CH, C), jnp.float32),
                      pltpu.VMEM((CH, C), jnp.float32))

    cp = pltpu.CompilerParams()
    if "needs_layout_passes" in pltpu.CompilerParams.__dataclass_fields__:
        cp = dataclasses.replace(cp, needs_layout_passes=False)

    return pl.kernel(
        body,
        out_type=jax.ShapeDtypeStruct((NPAD, C), jnp.float32),
        mesh=mesh,
        compiler_params=cp,
        scratch_types=[
            pltpu.VMEM((NPAD,), jnp.float32),
            pltpu.VMEM((K * NPT,), jnp.float32),
            pltpu.VMEM((K * NPT,), jnp.int32),
            pltpu.VMEM((K * NPT,), jnp.int32),
            pltpu.VMEM((K * NPT + LANES,), jnp.float32),
            pltpu.VMEM((K * NPT,), jnp.int32),
            pltpu.VMEM_SHARED((NT * NPAD,), jnp.float32),
            pltpu.VMEM_SHARED((2 * NT * NPAD,), jnp.int32),
            pltpu.VMEM_SHARED((NPAD,), jnp.float32),
            pltpu.SemaphoreType.DMA,
        ],
    )


def kernel(x, edge_index, W, att):
    N, IN = x.shape
    C = W.shape[0]
    E = edge_index.shape[1]

    NPAD = ((N + NT * LANES - 1) // (NT * LANES)) * NT * LANES
    EP = E + N
    EPT = ((EP + NT * LANES - 1) // (NT * LANES)) * LANES
    EPAD = EPT * NT

    WT = W.T
    attd = att[0, 0, :C].reshape(C, 1)
    atts = att[0, 0, C:].reshape(C, 1)

    xp, a1, b1 = _make_tc(N, IN, C)(x, WT, attd, atts)
    a = jnp.pad(a1[:, 0], (0, NPAD - N))
    b = jnp.pad(b1[:, 0], (0, NPAD - N))

    loops = jnp.arange(N, dtype=jnp.int32)
    rowp = jnp.concatenate(
        [edge_index[0], loops,
         jnp.full((EPAD - EP,), N, jnp.int32)])
    colp = jnp.concatenate(
        [edge_index[1], loops,
         jnp.zeros((EPAD - EP,), jnp.int32)])
    packed = rowp | (colp << RBITS)

    out = _make_sc(N, C, EPAD, NPAD)(packed, a, b, xp)
    return out[:N]


# speedup vs baseline: 1.3508x; 1.1354x over previous
import dataclasses
import functools

import jax
import jax.numpy as jnp
from jax import lax
from jax.experimental import pallas as pl
from jax.experimental.pallas import tpu as pltpu
from jax.experimental.pallas import tpu_sc as plsc

K = 5
LANES = 16
NT = 16
MINF = float("-inf")
RBITS = 14


def _tc_body(x_ref, wt_ref, ad_ref, as_ref, xp_ref, a_ref, b_ref):
    xp = jnp.dot(x_ref[...], wt_ref[...], preferred_element_type=jnp.float32)
    xp_ref[...] = xp
    a_ref[...] = jnp.dot(xp, ad_ref[...], preferred_element_type=jnp.float32,
                         precision=jax.lax.Precision.HIGHEST)
    b_ref[...] = jnp.dot(xp, as_ref[...], preferred_element_type=jnp.float32,
                         precision=jax.lax.Precision.HIGHEST)


@functools.lru_cache(maxsize=None)
def _make_tc(N, IN, C):
    BLK = 2000
    assert N % BLK == 0
    grid = (N // BLK,)
    return pl.pallas_call(
        _tc_body,
        grid=grid,
        in_specs=[
            pl.BlockSpec((BLK, IN), lambda i: (i, 0)),
            pl.BlockSpec((IN, C), lambda i: (0, 0)),
            pl.BlockSpec((C, 1), lambda i: (0, 0)),
            pl.BlockSpec((C, 1), lambda i: (0, 0)),
        ],
        out_specs=[
            pl.BlockSpec((BLK, C), lambda i: (i, 0)),
            pl.BlockSpec((BLK, 1), lambda i: (i, 0)),
            pl.BlockSpec((BLK, 1), lambda i: (i, 0)),
        ],
        out_shape=[
            jax.ShapeDtypeStruct((N, C), jnp.float32),
            jax.ShapeDtypeStruct((N, 1), jnp.float32),
            jax.ShapeDtypeStruct((N, 1), jnp.float32),
        ],
    )


@functools.lru_cache(maxsize=None)
def _make_sc(N, C, EPAD, NPAD):
    EPT = EPAD // NT
    NPT = NPAD // NT
    NV_E = EPT // LANES
    NV_N = NPT // LANES
    NV_P = NPAD // LANES
    CH = 16
    NCH = NPT // CH
    CL = C // LANES
    RMASK = (1 << RBITS) - 1

    mesh = plsc.VectorSubcoreMesh(
        core_axis_name="c", subcore_axis_name="s", num_cores=1,
        num_subcores=NT)

    def body(pe_hbm, a_hbm, b_hbm, xp_hbm, out_hbm,
             gm, mk, ck, colk, wk, selcol, sh16f, sh16i, spub, sem):
        tid = lax.axis_index("s")

        def fill(ref, nvec, val, dtype):
            def fb(i, _):
                ref[pl.ds(i * LANES, LANES)] = jnp.full((LANES,), val, dtype)
                return 0
            lax.fori_loop(0, nvec, fb, 0)

        def selection(pev, bnode):
            with jax.named_scope("prologue"):
                pltpu.sync_copy(pe_hbm.at[pl.ds(tid * EPT, EPT)], pev)
                pltpu.sync_copy(b_hbm, bnode)

            def rmw_max(rows, v, macc):
                g = plsc.load_gather(macc, [rows])
                plsc.store_scatter(macc, [rows], jnp.maximum(g, v),
                                   mask=v > g)
                g2 = plsc.load_gather(macc, [rows])

                @pl.when(jnp.any(v > g2))
                def _slow():
                    def cond(gc):
                        return jnp.any(v > gc)

                    def bdy(gc):
                        plsc.store_scatter(macc, [rows], jnp.maximum(gc, v),
                                           mask=v > gc)
                        return plsc.load_gather(macc, [rows])

                    lax.while_loop(cond, bdy, g2)

            def max_pass0(macc):
                def eb(i, _):
                    for u in range(2):
                        st = (2 * i + u) * LANES
                        pe = pev[pl.ds(st, LANES)]
                        rows = pe & RMASK
                        cols = lax.shift_right_logical(pe, RBITS)
                        be = plsc.load_gather(bnode, [cols])
                        rmw_max(rows, be, macc)
                    return 0
                lax.fori_loop(0, NV_E // 2, eb, 0)

            def fused_pass(cnt, colc, macc, with_max):
                ones = jnp.ones((LANES,), jnp.int32)

                def eb(i, _):
                    for u in range(2):
                        st = (2 * i + u) * LANES
                        pe = pev[pl.ds(st, LANES)]
                        rows = pe & RMASK
                        cols = lax.shift_right_logical(pe, RBITS)
                        be = plsc.load_gather(bnode, [cols])
                        mv = plsc.load_gather(gm, [rows])
                        eq = be == mv
                        plsc.addupdate_scatter(cnt, [rows], ones, mask=eq)
                        plsc.store_scatter(colc, [rows], cols, mask=eq)
                        if with_max:
                            v = jnp.where(
                                be < mv, be,
                                jnp.full((LANES,), MINF, jnp.float32))
                            rmw_max(rows, v, macc)
                    return 0
                lax.fori_loop(0, NV_E // 2, eb, 0)

            def merge_max(k, macc):
                stagef = macc
                pltpu.sync_copy(macc, sh16f.at[pl.ds(tid * NPAD, NPAD)])
                plsc.subcore_barrier()
                for j in range(NT):
                    pltpu.sync_copy(
                        sh16f.at[pl.ds(j * NPAD + tid * NPT, NPT)],
                        stagef.at[pl.ds(j * NPT, NPT)])

                def rb(v, _):
                    st = v * LANES
                    red = stagef[pl.ds(st, LANES)]
                    for j in range(1, NT):
                        red = jnp.maximum(red,
                                          stagef[pl.ds(j * NPT + st, LANES)])
                    mk[pl.ds(k * NPT + st, LANES)] = red
                    return 0
                lax.fori_loop(0, NV_N, rb, 0)
                plsc.subcore_barrier()
                pltpu.sync_copy(mk.at[pl.ds(k * NPT, NPT)],
                                spub.at[pl.ds(tid * NPT, NPT)])
                plsc.subcore_barrier()
                pltpu.sync_copy(spub, gm)
                plsc.subcore_barrier()

            def merge_cnts(k, cnt, colc):
                stagei = cnt
                pltpu.sync_copy(cnt, sh16i.at[pl.ds(tid * 2 * NPAD, NPAD)])
                pltpu.sync_copy(colc,
                                sh16i.at[pl.ds(tid * 2 * NPAD + NPAD, NPAD)])
                plsc.subcore_barrier()
                for j in range(NT):
                    pltpu.sync_copy(
                        sh16i.at[pl.ds(2 * j * NPAD + tid * NPT, NPT)],
                        stagei.at[pl.ds(j * NPT, NPT)])

                def rbc(v, _):
                    st = v * LANES
                    redc = stagei[pl.ds(st, LANES)]
                    for j in range(1, NT):
                        redc = redc + stagei[pl.ds(j * NPT + st, LANES)]
                    ck[pl.ds(k * NPT + st, LANES)] = redc
                    return 0
                lax.fori_loop(0, NV_N, rbc, 0)
                for j in range(NT):
                    pltpu.sync_copy(
                        sh16i.at[pl.ds((2 * j + 1) * NPAD + tid * NPT, NPT)],
                        stagei.at[pl.ds(j * NPT, NPT)])

                def rbl(v, _):
                    st = v * LANES
                    redl = stagei[pl.ds(st, LANES)]
                    for j in range(1, NT):
                        redl = jnp.maximum(redl,
                                           stagei[pl.ds(j * NPT + st, LANES)])
                    colk[pl.ds(k * NPT + st, LANES)] = redl
                    return 0
                lax.fori_loop(0, NV_N, rbl, 0)
                plsc.subcore_barrier()

            def rounds(cnt, colc, macc):
                with jax.named_scope("p0_fill"):
                    fill(macc, NV_P, MINF, jnp.float32)
                with jax.named_scope("p0_max"):
                    max_pass0(macc)
                with jax.named_scope("p0_merge"):
                    merge_max(0, macc)
                for k in range(1, K + 1):
                    with_max = k < K
                    with jax.named_scope(f"r{k}_fill"):
                        def fb(i, _, _wm=with_max):
                            sl = pl.ds(i * LANES, LANES)
                            cnt[sl] = jnp.zeros((LANES,), jnp.int32)
                            colc[sl] = jnp.full((LANES,), -1, jnp.int32)
                            if _wm:
                                macc[sl] = jnp.full((LANES,), MINF,
                                                    jnp.float32)
                            return 0
                        lax.fori_loop(0, NV_P, fb, 0)
                    with jax.named_scope(f"r{k}_pass"):
                        fused_pass(cnt, colc, macc, with_max)
                    with jax.named_scope(f"r{k}_mcnt"):
                        merge_cnts(k - 1, cnt, colc)
                    if with_max:
                        with jax.named_scope(f"r{k}_mmax"):
                            merge_max(k, macc)
            pl.run_scoped(rounds,
                          pltpu.VMEM((NPAD,), jnp.int32),
                          pltpu.VMEM((NPAD,), jnp.int32),
                          pltpu.VMEM((NPAD,), jnp.float32))

        pl.run_scoped(selection,
                      pltpu.VMEM((EPT,), jnp.int32),
                      pltpu.VMEM((NPAD,), jnp.float32))

        def weights(anode):
            pltpu.sync_copy(a_hbm.at[pl.ds(tid * NPT, NPT)], anode)

            def wb(v, _):
                st = v * LANES
                a16 = anode[pl.ds(st, LANES)]
                rem = jnp.full((LANES,), K, jnp.int32)
                denom = jnp.zeros((LANES,), jnp.float32)
                exs = []
                s0 = None
                for k in range(K):
                    m = mk[pl.ds(k * NPT + st, LANES)]
                    c = ck[pl.ds(k * NPT + st, LANES)]
                    s = a16 + m
                    s = jnp.where(s >= 0, s, 0.2 * s)
                    if k == 0:
                        s0 = s
                    t = jnp.minimum(c, jnp.maximum(rem, 0))
                    rem = rem - t
                    valid = t > 0
                    ex = jnp.where(valid,
                                   t.astype(jnp.float32) * jnp.exp(s - s0),
                                   jnp.zeros((LANES,), jnp.float32))
                    denom = denom + ex
                    exs.append(ex)
                    cv = colk[pl.ds(k * NPT + st, LANES)]
                    selcol[pl.ds(k * NPT + st, LANES)] = jnp.where(
                        cv >= 0, cv, jnp.zeros((LANES,), jnp.int32))
                for k in range(K):
                    wk[pl.ds(k * NPT + st, LANES)] = exs[k] / denom
                return 0
            lax.fori_loop(0, NV_N, wb, 0)
        with jax.named_scope("weights"):
            pl.run_scoped(weights, pltpu.VMEM((NPT,), jnp.float32))

        def aggregate(rowbuf, outchunk):
            def fire(ch, buf):
                descs = []
                for k in range(K):
                    d = pltpu.async_copy(
                        xp_hbm.at[selcol.at[pl.ds(k * NPT + ch * CH, CH)]],
                        rowbuf.at[pl.ds((buf * K + k) * CH, CH)], sem)
                    descs.append(d)
                return descs

            pending = fire(0, 0)
            for ch in range(NCH):
                buf = ch % 2
                for d in pending:
                    d.wait()
                if ch + 1 < NCH:
                    pending = fire(ch + 1, 1 - buf)

                def nbody(n, _):
                    zero = jnp.zeros((LANES,), jnp.float32)
                    accs0 = tuple(zero for _ in range(CL))

                    def kbody(k, accs):
                        wv = wk[pl.ds(k * NPT + ch * CH + n, LANES)]
                        wb16 = jnp.full((LANES,), wv[0])
                        out = []
                        for c in range(CL):
                            r = rowbuf[(buf * K + k) * CH + n,
                                       pl.ds(c * LANES, LANES)]
                            out.append(accs[c] + wb16 * r)
                        return tuple(out)

                    accs = lax.fori_loop(0, K, kbody, accs0)
                    for c in range(CL):
                        v = accs[c]
                        ev = jnp.where(
                            v > 0, v,
                            jnp.exp(jnp.minimum(v, 0.0)) - 1.0)
                        outchunk[n, pl.ds(c * LANES, LANES)] = ev
                    return 0
                lax.fori_loop(0, CH, nbody, 0)
                pltpu.sync_copy(outchunk,
                                out_hbm.at[pl.ds(tid * NPT + ch * CH, CH), :])
        with jax.named_scope("agg"):
            pl.run_scoped(aggregate,
                          pltpu.VMEM((2 * K * CH, C), jnp.float32),
                          pltpu.VMEM((CH, C), jnp.float32))

    cp = pltpu.CompilerParams()
    if "needs_layout_passes" in pltpu.CompilerParams.__dataclass_fields__:
        cp = dataclasses.replace(cp, needs_layout_passes=False)

    return pl.kernel(
        body,
        out_type=jax.ShapeDtypeStruct((NPAD, C), jnp.float32),
        mesh=mesh,
        compiler_params=cp,
        scratch_types=[
            pltpu.VMEM((NPAD,), jnp.float32),
            pltpu.VMEM((K * NPT,), jnp.float32),
            pltpu.VMEM((K * NPT,), jnp.int32),
            pltpu.VMEM((K * NPT,), jnp.int32),
            pltpu.VMEM((K * NPT + LANES,), jnp.float32),
            pltpu.VMEM((K * NPT,), jnp.int32),
            pltpu.VMEM_SHARED((NT * NPAD,), jnp.float32),
            pltpu.VMEM_SHARED((2 * NT * NPAD,), jnp.int32),
            pltpu.VMEM_SHARED((NPAD,), jnp.float32),
            pltpu.SemaphoreType.DMA,
        ],
    )


def kernel(x, edge_index, W, att):
    N, IN = x.shape
    C = W.shape[0]
    E = edge_index.shape[1]

    NPAD = ((N + NT * LANES - 1) // (NT * LANES)) * NT * LANES
    EP = E + N
    EPT = ((EP + NT * LANES - 1) // (NT * LANES)) * LANES
    EPAD = EPT * NT

    WT = W.T
    attd = att[0, 0, :C].reshape(C, 1)
    atts = att[0, 0, C:].reshape(C, 1)

    xp, a1, b1 = _make_tc(N, IN, C)(x, WT, attd, atts)
    a = jnp.pad(a1[:, 0], (0, NPAD - N))
    b = jnp.pad(b1[:, 0], (0, NPAD - N))

    loops = jnp.arange(N, dtype=jnp.int32)
    rowp = jnp.concatenate(
        [edge_index[0], loops,
         jnp.full((EPAD - EP,), N, jnp.int32)])
    colp = jnp.concatenate(
        [edge_index[1], loops,
         jnp.zeros((EPAD - EP,), jnp.int32)])
    packed = rowp | (colp << RBITS)

    out = _make_sc(N, C, EPAD, NPAD)(packed, a, b, xp)
    return out[:N]


# speedup vs baseline: 1.6236x; 1.2019x over previous
import dataclasses
import functools

import jax
import jax.numpy as jnp
from jax import lax
from jax.experimental import pallas as pl
from jax.experimental.pallas import tpu as pltpu
from jax.experimental.pallas import tpu_sc as plsc

K = 5
LANES = 16
NT = 16
MINF = float("-inf")
RBITS = 14


def _tc_body(x_ref, wt_ref, ad_ref, as_ref, xp_ref, a_ref, b_ref):
    xp = jnp.dot(x_ref[...], wt_ref[...], preferred_element_type=jnp.float32)
    xp_ref[...] = xp
    a_ref[...] = jnp.dot(xp, ad_ref[...], preferred_element_type=jnp.float32,
                         precision=jax.lax.Precision.HIGHEST)
    b_ref[...] = jnp.dot(xp, as_ref[...], preferred_element_type=jnp.float32,
                         precision=jax.lax.Precision.HIGHEST)


@functools.lru_cache(maxsize=None)
def _make_tc(N, IN, C):
    BLK = 2000
    assert N % BLK == 0
    grid = (N // BLK,)
    return pl.pallas_call(
        _tc_body,
        grid=grid,
        in_specs=[
            pl.BlockSpec((BLK, IN), lambda i: (i, 0)),
            pl.BlockSpec((IN, C), lambda i: (0, 0)),
            pl.BlockSpec((C, 1), lambda i: (0, 0)),
            pl.BlockSpec((C, 1), lambda i: (0, 0)),
        ],
        out_specs=[
            pl.BlockSpec((BLK, C), lambda i: (i, 0)),
            pl.BlockSpec((BLK, 1), lambda i: (i, 0)),
            pl.BlockSpec((BLK, 1), lambda i: (i, 0)),
        ],
        out_shape=[
            jax.ShapeDtypeStruct((N, C), jnp.float32),
            jax.ShapeDtypeStruct((N, 1), jnp.float32),
            jax.ShapeDtypeStruct((N, 1), jnp.float32),
        ],
    )


@functools.lru_cache(maxsize=None)
def _make_sc(N, C, EPAD, NPAD):
    EPT = EPAD // NT
    NPT = NPAD // NT
    NV_E = EPT // LANES
    NV_N = NPT // LANES
    NV_P = NPAD // LANES
    CH = 16
    NCH = NPT // CH
    CL = C // LANES
    RMASK = (1 << RBITS) - 1

    mesh = plsc.VectorSubcoreMesh(
        core_axis_name="c", subcore_axis_name="s", num_cores=1,
        num_subcores=NT)

    def body(pe_hbm, a_hbm, b_hbm, xp_hbm, out_hbm,
             gm, mk, ck, colk, wk, selcol, sh16f, sh16i, spub, sem):
        tid = lax.axis_index("s")

        def fill(ref, nvec, val, dtype):
            def fb(i, _):
                ref[pl.ds(i * LANES, LANES)] = jnp.full((LANES,), val, dtype)
                return 0
            lax.fori_loop(0, nvec, fb, 0)

        def selection(pev, bnode):
            with jax.named_scope("prologue"):
                pltpu.sync_copy(pe_hbm.at[pl.ds(tid * EPT, EPT)], pev)
                pltpu.sync_copy(b_hbm, bnode)

            def _fix(rows, v, macc):
                def cond(gc):
                    return jnp.any(v > gc)

                def bdy(gc):
                    plsc.store_scatter(macc, [rows], jnp.maximum(gc, v),
                                       mask=v > gc)
                    return plsc.load_gather(macc, [rows])

                lax.while_loop(cond, bdy, plsc.load_gather(macc, [rows]))

            def rmw_max_pair(pairs, macc):
                for rows, v in pairs:
                    g = plsc.load_gather(macc, [rows])
                    plsc.store_scatter(macc, [rows], jnp.maximum(g, v),
                                       mask=v > g)
                lost = None
                for rows, v in pairs:
                    gv = plsc.load_gather(macc, [rows])
                    l = v > gv
                    lost = l if lost is None else (lost | l)

                @pl.when(jnp.any(lost))
                def _slow():
                    for rows, v in pairs:
                        _fix(rows, v, macc)

            def max_pass0(macc):
                def eb(i, _):
                    pairs = []
                    for u in range(2):
                        st = (2 * i + u) * LANES
                        pe = pev[pl.ds(st, LANES)]
                        rows = pe & RMASK
                        cols = lax.shift_right_logical(pe, RBITS)
                        be = plsc.load_gather(bnode, [cols])
                        pairs.append((rows, be))
                    rmw_max_pair(pairs, macc)
                    return 0
                lax.fori_loop(0, NV_E // 2, eb, 0)

            def fused_pass(cnt, colc, macc, with_max):
                ones = jnp.ones((LANES,), jnp.int32)

                def eb(i, _):
                    pairs = []
                    for u in range(2):
                        st = (2 * i + u) * LANES
                        pe = pev[pl.ds(st, LANES)]
                        rows = pe & RMASK
                        cols = lax.shift_right_logical(pe, RBITS)
                        be = plsc.load_gather(bnode, [cols])
                        mv = plsc.load_gather(gm, [rows])
                        eq = be == mv
                        plsc.addupdate_scatter(cnt, [rows], ones, mask=eq)
                        plsc.store_scatter(colc, [rows], cols, mask=eq)
                        if with_max:
                            v = jnp.where(
                                be < mv, be,
                                jnp.full((LANES,), MINF, jnp.float32))
                            pairs.append((rows, v))
                    if with_max:
                        rmw_max_pair(pairs, macc)
                    return 0
                lax.fori_loop(0, NV_E // 2, eb, 0)

            def merge_max(k, macc):
                stagef = macc
                pltpu.sync_copy(macc, sh16f.at[pl.ds(tid * NPAD, NPAD)])
                plsc.subcore_barrier()
                for j in range(NT):
                    pltpu.sync_copy(
                        sh16f.at[pl.ds(j * NPAD + tid * NPT, NPT)],
                        stagef.at[pl.ds(j * NPT, NPT)])

                def rb(v, _):
                    st = v * LANES
                    red = stagef[pl.ds(st, LANES)]
                    for j in range(1, NT):
                        red = jnp.maximum(red,
                                          stagef[pl.ds(j * NPT + st, LANES)])
                    mk[pl.ds(k * NPT + st, LANES)] = red
                    return 0
                lax.fori_loop(0, NV_N, rb, 0)
                plsc.subcore_barrier()
                pltpu.sync_copy(mk.at[pl.ds(k * NPT, NPT)],
                                spub.at[pl.ds(tid * NPT, NPT)])
                plsc.subcore_barrier()
                pltpu.sync_copy(spub, gm)
                plsc.subcore_barrier()

            def merge_cnts(k, cnt, colc):
                stagei = cnt
                pltpu.sync_copy(cnt, sh16i.at[pl.ds(tid * 2 * NPAD, NPAD)])
                pltpu.sync_copy(colc,
                                sh16i.at[pl.ds(tid * 2 * NPAD + NPAD, NPAD)])
                plsc.subcore_barrier()
                for j in range(NT):
                    pltpu.sync_copy(
                        sh16i.at[pl.ds(2 * j * NPAD + tid * NPT, NPT)],
                        stagei.at[pl.ds(j * NPT, NPT)])

                def rbc(v, _):
                    st = v * LANES
                    redc = stagei[pl.ds(st, LANES)]
                    for j in range(1, NT):
                        redc = redc + stagei[pl.ds(j * NPT + st, LANES)]
                    ck[pl.ds(k * NPT + st, LANES)] = redc
                    return 0
                lax.fori_loop(0, NV_N, rbc, 0)
                for j in range(NT):
                    pltpu.sync_copy(
                        sh16i.at[pl.ds((2 * j + 1) * NPAD + tid * NPT, NPT)],
                        stagei.at[pl.ds(j * NPT, NPT)])

                def rbl(v, _):
                    st = v * LANES
                    redl = stagei[pl.ds(st, LANES)]
                    for j in range(1, NT):
                        redl = jnp.maximum(redl,
                                           stagei[pl.ds(j * NPT + st, LANES)])
                    colk[pl.ds(k * NPT + st, LANES)] = redl
                    return 0
                lax.fori_loop(0, NV_N, rbl, 0)
                plsc.subcore_barrier()

            def rounds(cnt, colc, macc):
                with jax.named_scope("p0_fill"):
                    fill(macc, NV_P, MINF, jnp.float32)
                with jax.named_scope("p0_max"):
                    max_pass0(macc)
                with jax.named_scope("p0_merge"):
                    merge_max(0, macc)
                for k in range(1, K + 1):
                    with_max = k < K
                    with jax.named_scope(f"r{k}_fill"):
                        def fb(i, _, _wm=with_max):
                            sl = pl.ds(i * LANES, LANES)
                            cnt[sl] = jnp.zeros((LANES,), jnp.int32)
                            colc[sl] = jnp.full((LANES,), -1, jnp.int32)
                            if _wm:
                                macc[sl] = jnp.full((LANES,), MINF,
                                                    jnp.float32)
                            return 0
                        lax.fori_loop(0, NV_P, fb, 0)
                    with jax.named_scope(f"r{k}_pass"):
                        fused_pass(cnt, colc, macc, with_max)
                    with jax.named_scope(f"r{k}_mcnt"):
                        merge_cnts(k - 1, cnt, colc)
                    if with_max:
                        with jax.named_scope(f"r{k}_mmax"):
                            merge_max(k, macc)
            pl.run_scoped(rounds,
                          pltpu.VMEM((NPAD,), jnp.int32),
                          pltpu.VMEM((NPAD,), jnp.int32),
                          pltpu.VMEM((NPAD,), jnp.float32))

        pl.run_scoped(selection,
                      pltpu.VMEM((EPT,), jnp.int32),
                      pltpu.VMEM((NPAD,), jnp.float32))

        def weights(anode):
            pltpu.sync_copy(a_hbm.at[pl.ds(tid * NPT, NPT)], anode)

            def wb(v, _):
                st = v * LANES
                a16 = anode[pl.ds(st, LANES)]
                rem = jnp.full((LANES,), K, jnp.int32)
                denom = jnp.zeros((LANES,), jnp.float32)
                exs = []
                s0 = None
                for k in range(K):
                    m = mk[pl.ds(k * NPT + st, LANES)]
                    c = ck[pl.ds(k * NPT + st, LANES)]
                    s = a16 + m
                    s = jnp.where(s >= 0, s, 0.2 * s)
                    if k == 0:
                        s0 = s
                    t = jnp.minimum(c, jnp.maximum(rem, 0))
                    rem = rem - t
                    valid = t > 0
                    ex = jnp.where(valid,
                                   t.astype(jnp.float32) * jnp.exp(s - s0),
                                   jnp.zeros((LANES,), jnp.float32))
                    denom = denom + ex
                    exs.append(ex)
                    cv = colk[pl.ds(k * NPT + st, LANES)]
                    selcol[pl.ds(k * NPT + st, LANES)] = jnp.where(
                        cv >= 0, cv, jnp.zeros((LANES,), jnp.int32))
                for k in range(K):
                    wk[pl.ds(k * NPT + st, LANES)] = exs[k] / denom
                return 0
            lax.fori_loop(0, NV_N, wb, 0)
        with jax.named_scope("weights"):
            pl.run_scoped(weights, pltpu.VMEM((NPT,), jnp.float32))

        def aggregate(rowbuf, outchunk):
            def fire(ch, buf):
                descs = []
                for k in range(K):
                    d = pltpu.async_copy(
                        xp_hbm.at[selcol.at[pl.ds(k * NPT + ch * CH, CH)]],
                        rowbuf.at[pl.ds((buf * K + k) * CH, CH)], sem)
                    descs.append(d)
                return descs

            pending = fire(0, 0)
            for ch in range(NCH):
                buf = ch % 2
                for d in pending:
                    d.wait()
                if ch + 1 < NCH:
                    pending = fire(ch + 1, 1 - buf)

                def nbody(n, _):
                    zero = jnp.zeros((LANES,), jnp.float32)
                    accs0 = tuple(zero for _ in range(CL))

                    def kbody(k, accs):
                        wv = wk[pl.ds(k * NPT + ch * CH + n, LANES)]
                        wb16 = jnp.full((LANES,), wv[0])
                        out = []
                        for c in range(CL):
                            r = rowbuf[(buf * K + k) * CH + n,
                                       pl.ds(c * LANES, LANES)]
                            out.append(accs[c] + wb16 * r)
                        return tuple(out)

                    accs = lax.fori_loop(0, K, kbody, accs0)
                    for c in range(CL):
                        v = accs[c]
                        ev = jnp.where(
                            v > 0, v,
                            jnp.exp(jnp.minimum(v, 0.0)) - 1.0)
                        outchunk[n, pl.ds(c * LANES, LANES)] = ev
                    return 0
                lax.fori_loop(0, CH, nbody, 0)
                pltpu.sync_copy(outchunk,
                                out_hbm.at[pl.ds(tid * NPT + ch * CH, CH), :])
        with jax.named_scope("agg"):
            pl.run_scoped(aggregate,
                          pltpu.VMEM((2 * K * CH, C), jnp.float32),
                          pltpu.VMEM((CH, C), jnp.float32))

    cp = pltpu.CompilerParams()
    if "needs_layout_passes" in pltpu.CompilerParams.__dataclass_fields__:
        cp = dataclasses.replace(cp, needs_layout_passes=False)

    return pl.kernel(
        body,
        out_type=jax.ShapeDtypeStruct((NPAD, C), jnp.float32),
        mesh=mesh,
        compiler_params=cp,
        scratch_types=[
            pltpu.VMEM((NPAD,), jnp.float32),
            pltpu.VMEM((K * NPT,), jnp.float32),
            pltpu.VMEM((K * NPT,), jnp.int32),
            pltpu.VMEM((K * NPT,), jnp.int32),
            pltpu.VMEM((K * NPT + LANES,), jnp.float32),
            pltpu.VMEM((K * NPT,), jnp.int32),
            pltpu.VMEM_SHARED((NT * NPAD,), jnp.float32),
            pltpu.VMEM_SHARED((2 * NT * NPAD,), jnp.int32),
            pltpu.VMEM_SHARED((NPAD,), jnp.float32),
            pltpu.SemaphoreType.DMA,
        ],
    )


def kernel(x, edge_index, W, att):
    N, IN = x.shape
    C = W.shape[0]
    E = edge_index.shape[1]

    NPAD = ((N + NT * LANES - 1) // (NT * LANES)) * NT * LANES
    EP = E + N
    EPT = ((EP + NT * LANES - 1) // (NT * LANES)) * LANES
    EPAD = EPT * NT

    WT = W.T
    attd = att[0, 0, :C].reshape(C, 1)
    atts = att[0, 0, C:].reshape(C, 1)

    xp, a1, b1 = _make_tc(N, IN, C)(x, WT, attd, atts)
    a = jnp.pad(a1[:, 0], (0, NPAD - N))
    b = jnp.pad(b1[:, 0], (0, NPAD - N))

    loops = jnp.arange(N, dtype=jnp.int32)
    rowp = jnp.concatenate(
        [edge_index[0], loops,
         jnp.full((EPAD - EP,), N, jnp.int32)])
    colp = jnp.concatenate(
        [edge_index[1], loops,
         jnp.zeros((EPAD - EP,), jnp.int32)])
    packed = rowp | (colp << RBITS)

    out = _make_sc(N, C, EPAD, NPAD)(packed, a, b, xp)
    return out[:N]


# speedup vs baseline: 1.7615x; 1.0850x over previous
import dataclasses
import functools

import jax
import jax.numpy as jnp
from jax import lax
from jax.experimental import pallas as pl
from jax.experimental.pallas import tpu as pltpu
from jax.experimental.pallas import tpu_sc as plsc

K = 5
LANES = 16
NT = 16
MINF = float("-inf")
RBITS = 14


def _tc_body(x_ref, wt_ref, ad_ref, as_ref, xp_ref, a_ref, b_ref):
    xp = jnp.dot(x_ref[...], wt_ref[...], preferred_element_type=jnp.float32)
    xp_ref[...] = xp
    a_ref[...] = jnp.dot(xp, ad_ref[...], preferred_element_type=jnp.float32,
                         precision=jax.lax.Precision.HIGHEST)
    b_ref[...] = jnp.dot(xp, as_ref[...], preferred_element_type=jnp.float32,
                         precision=jax.lax.Precision.HIGHEST)


@functools.lru_cache(maxsize=None)
def _make_tc(N, IN, C):
    BLK = 2000
    assert N % BLK == 0
    grid = (N // BLK,)
    return pl.pallas_call(
        _tc_body,
        grid=grid,
        in_specs=[
            pl.BlockSpec((BLK, IN), lambda i: (i, 0)),
            pl.BlockSpec((IN, C), lambda i: (0, 0)),
            pl.BlockSpec((C, 1), lambda i: (0, 0)),
            pl.BlockSpec((C, 1), lambda i: (0, 0)),
        ],
        out_specs=[
            pl.BlockSpec((BLK, C), lambda i: (i, 0)),
            pl.BlockSpec((BLK, 1), lambda i: (i, 0)),
            pl.BlockSpec((BLK, 1), lambda i: (i, 0)),
        ],
        out_shape=[
            jax.ShapeDtypeStruct((N, C), jnp.float32),
            jax.ShapeDtypeStruct((N, 1), jnp.float32),
            jax.ShapeDtypeStruct((N, 1), jnp.float32),
        ],
    )


@functools.lru_cache(maxsize=None)
def _make_sc(N, C, EPAD, NPAD):
    EPT = EPAD // NT
    NPT = NPAD // NT
    NV_E = EPT // LANES
    NV_N = NPT // LANES
    NV_P = NPAD // LANES
    CH = 16
    NCH = NPT // CH
    CL = C // LANES
    RMASK = (1 << RBITS) - 1

    mesh = plsc.VectorSubcoreMesh(
        core_axis_name="c", subcore_axis_name="s", num_cores=1,
        num_subcores=NT)

    def body(pe_hbm, a_hbm, b_hbm, xp_hbm, out_hbm,
             gm, mk, ck, colk, wk, selcol, sh16f, sh16i, spub, sem):
        tid = lax.axis_index("s")

        def fill(ref, nvec, val, dtype):
            def fb(i, _):
                ref[pl.ds(i * LANES, LANES)] = jnp.full((LANES,), val, dtype)
                return 0
            lax.fori_loop(0, nvec, fb, 0)

        def selection(pev, bnode):
            with jax.named_scope("prologue"):
                pltpu.sync_copy(pe_hbm.at[pl.ds(tid * EPT, EPT)], pev)
                pltpu.sync_copy(b_hbm, bnode)

            def _fix(rows, v, macc):
                def cond(gc):
                    return jnp.any(v > gc)

                def bdy(gc):
                    plsc.store_scatter(macc, [rows], jnp.maximum(gc, v),
                                       mask=v > gc)
                    return plsc.load_gather(macc, [rows])

                lax.while_loop(cond, bdy, plsc.load_gather(macc, [rows]))

            def rmw_max_pair(pairs, macc):
                for rows, v in pairs:
                    g = plsc.load_gather(macc, [rows])
                    plsc.store_scatter(macc, [rows], jnp.maximum(g, v),
                                       mask=v > g)
                lost = None
                for rows, v in pairs:
                    gv = plsc.load_gather(macc, [rows])
                    l = v > gv
                    lost = l if lost is None else (lost | l)

                @pl.when(jnp.any(lost))
                def _slow():
                    for rows, v in pairs:
                        _fix(rows, v, macc)

            def max_pass0(macc):
                def eb(i, _):
                    pairs = []
                    for u in range(4):
                        st = (4 * i + u) * LANES
                        pe = pev[pl.ds(st, LANES)]
                        rows = pe & RMASK
                        cols = lax.shift_right_logical(pe, RBITS)
                        be = plsc.load_gather(bnode, [cols])
                        pairs.append((rows, be))
                    rmw_max_pair(pairs, macc)
                    return 0
                lax.fori_loop(0, NV_E // 4, eb, 0)

            def fused_pass(cnt, colc, macc, with_max):
                ones = jnp.ones((LANES,), jnp.int32)

                def eb(i, _):
                    pairs = []
                    for u in range(4):
                        st = (4 * i + u) * LANES
                        pe = pev[pl.ds(st, LANES)]
                        rows = pe & RMASK
                        cols = lax.shift_right_logical(pe, RBITS)
                        be = plsc.load_gather(bnode, [cols])
                        mv = plsc.load_gather(gm, [rows])
                        eq = be == mv
                        plsc.addupdate_scatter(cnt, [rows], ones, mask=eq)
                        plsc.store_scatter(colc, [rows], cols, mask=eq)
                        if with_max:
                            v = jnp.where(
                                be < mv, be,
                                jnp.full((LANES,), MINF, jnp.float32))
                            pairs.append((rows, v))
                    if with_max:
                        rmw_max_pair(pairs, macc)
                    return 0
                lax.fori_loop(0, NV_E // 4, eb, 0)

            def merge_max(k, macc):
                stagef = macc
                pltpu.sync_copy(macc, sh16f.at[pl.ds(tid * NPAD, NPAD)])
                plsc.subcore_barrier()
                for j in range(NT):
                    pltpu.sync_copy(
                        sh16f.at[pl.ds(j * NPAD + tid * NPT, NPT)],
                        stagef.at[pl.ds(j * NPT, NPT)])

                def rb(v, _):
                    st = v * LANES
                    red = stagef[pl.ds(st, LANES)]
                    for j in range(1, NT):
                        red = jnp.maximum(red,
                                          stagef[pl.ds(j * NPT + st, LANES)])
                    mk[pl.ds(k * NPT + st, LANES)] = red
                    return 0
                lax.fori_loop(0, NV_N, rb, 0)
                plsc.subcore_barrier()
                pltpu.sync_copy(mk.at[pl.ds(k * NPT, NPT)],
                                spub.at[pl.ds(tid * NPT, NPT)])
                plsc.subcore_barrier()
                pltpu.sync_copy(spub, gm)
                plsc.subcore_barrier()

            def merge_cnts(k, cnt, colc):
                stagei = cnt
                pltpu.sync_copy(cnt, sh16i.at[pl.ds(tid * 2 * NPAD, NPAD)])
                pltpu.sync_copy(colc,
                                sh16i.at[pl.ds(tid * 2 * NPAD + NPAD, NPAD)])
                plsc.subcore_barrier()
                for j in range(NT):
                    pltpu.sync_copy(
                        sh16i.at[pl.ds(2 * j * NPAD + tid * NPT, NPT)],
                        stagei.at[pl.ds(j * NPT, NPT)])

                def rbc(v, _):
                    st = v * LANES
                    redc = stagei[pl.ds(st, LANES)]
                    for j in range(1, NT):
                        redc = redc + stagei[pl.ds(j * NPT + st, LANES)]
                    ck[pl.ds(k * NPT + st, LANES)] = redc
                    return 0
                lax.fori_loop(0, NV_N, rbc, 0)
                for j in range(NT):
                    pltpu.sync_copy(
                        sh16i.at[pl.ds((2 * j + 1) * NPAD + tid * NPT, NPT)],
                        stagei.at[pl.ds(j * NPT, NPT)])

                def rbl(v, _):
                    st = v * LANES
                    redl = stagei[pl.ds(st, LANES)]
                    for j in range(1, NT):
                        redl = jnp.maximum(redl,
                                           stagei[pl.ds(j * NPT + st, LANES)])
                    colk[pl.ds(k * NPT + st, LANES)] = redl
                    return 0
                lax.fori_loop(0, NV_N, rbl, 0)
                plsc.subcore_barrier()

            def rounds(cnt, colc, macc):
                with jax.named_scope("p0_fill"):
                    fill(macc, NV_P, MINF, jnp.float32)
                with jax.named_scope("p0_max"):
                    max_pass0(macc)
                with jax.named_scope("p0_merge"):
                    merge_max(0, macc)
                for k in range(1, K + 1):
                    with_max = k < K
                    with jax.named_scope(f"r{k}_fill"):
                        def fb(i, _, _wm=with_max):
                            sl = pl.ds(i * LANES, LANES)
                            cnt[sl] = jnp.zeros((LANES,), jnp.int32)
                            colc[sl] = jnp.full((LANES,), -1, jnp.int32)
                            if _wm:
                                macc[sl] = jnp.full((LANES,), MINF,
                                                    jnp.float32)
                            return 0
                        lax.fori_loop(0, NV_P, fb, 0)
                    with jax.named_scope(f"r{k}_pass"):
                        fused_pass(cnt, colc, macc, with_max)
                    with jax.named_scope(f"r{k}_mcnt"):
                        merge_cnts(k - 1, cnt, colc)
                    if with_max:
                        with jax.named_scope(f"r{k}_mmax"):
                            merge_max(k, macc)
            pl.run_scoped(rounds,
                          pltpu.VMEM((NPAD,), jnp.int32),
                          pltpu.VMEM((NPAD,), jnp.int32),
                          pltpu.VMEM((NPAD,), jnp.float32))

        pl.run_scoped(selection,
                      pltpu.VMEM((EPT,), jnp.int32),
                      pltpu.VMEM((NPAD,), jnp.float32))

        def weights(anode):
            pltpu.sync_copy(a_hbm.at[pl.ds(tid * NPT, NPT)], anode)

            def wb(v, _):
                st = v * LANES
                a16 = anode[pl.ds(st, LANES)]
                rem = jnp.full((LANES,), K, jnp.int32)
                denom = jnp.zeros((LANES,), jnp.float32)
                exs = []
                s0 = None
                for k in range(K):
                    m = mk[pl.ds(k * NPT + st, LANES)]
                    c = ck[pl.ds(k * NPT + st, LANES)]
                    s = a16 + m
                    s = jnp.where(s >= 0, s, 0.2 * s)
                    if k == 0:
                        s0 = s
                    t = jnp.minimum(c, jnp.maximum(rem, 0))
                    rem = rem - t
                    valid = t > 0
                    ex = jnp.where(valid,
                                   t.astype(jnp.float32) * jnp.exp(s - s0),
                                   jnp.zeros((LANES,), jnp.float32))
                    denom = denom + ex
                    exs.append(ex)
                    cv = colk[pl.ds(k * NPT + st, LANES)]
                    selcol[pl.ds(k * NPT + st, LANES)] = jnp.where(
                        cv >= 0, cv, jnp.zeros((LANES,), jnp.int32))
                for k in range(K):
                    wk[pl.ds(k * NPT + st, LANES)] = exs[k] / denom
                return 0
            lax.fori_loop(0, NV_N, wb, 0)
        with jax.named_scope("weights"):
            pl.run_scoped(weights, pltpu.VMEM((NPT,), jnp.float32))

        def aggregate(rowbuf, outchunk):
            def fire(ch, buf):
                descs = []
                for k in range(K):
                    d = pltpu.async_copy(
                        xp_hbm.at[selcol.at[pl.ds(k * NPT + ch * CH, CH)]],
                        rowbuf.at[pl.ds((buf * K + k) * CH, CH)], sem)
                    descs.append(d)
                return descs

            pending = fire(0, 0)
            for ch in range(NCH):
                buf = ch % 2
                for d in pending:
                    d.wait()
                if ch + 1 < NCH:
                    pending = fire(ch + 1, 1 - buf)

                def nbody(n, _):
                    zero = jnp.zeros((LANES,), jnp.float32)
                    accs0 = tuple(zero for _ in range(CL))

                    def kbody(k, accs):
                        wv = wk[pl.ds(k * NPT + ch * CH + n, LANES)]
                        wb16 = jnp.full((LANES,), wv[0])
                        out = []
                        for c in range(CL):
                            r = rowbuf[(buf * K + k) * CH + n,
                                       pl.ds(c * LANES, LANES)]
                            out.append(accs[c] + wb16 * r)
                        return tuple(out)

                    accs = lax.fori_loop(0, K, kbody, accs0)
                    for c in range(CL):
                        v = accs[c]
                        ev = jnp.where(
                            v > 0, v,
                            jnp.exp(jnp.minimum(v, 0.0)) - 1.0)
                        outchunk[n, pl.ds(c * LANES, LANES)] = ev
                    return 0
                lax.fori_loop(0, CH, nbody, 0)
                pltpu.sync_copy(outchunk,
                                out_hbm.at[pl.ds(tid * NPT + ch * CH, CH), :])
        with jax.named_scope("agg"):
            pl.run_scoped(aggregate,
                          pltpu.VMEM((2 * K * CH, C), jnp.float32),
                          pltpu.VMEM((CH, C), jnp.float32))

    cp = pltpu.CompilerParams()
    if "needs_layout_passes" in pltpu.CompilerParams.__dataclass_fields__:
        cp = dataclasses.replace(cp, needs_layout_passes=False)

    return pl.kernel(
        body,
        out_type=jax.ShapeDtypeStruct((NPAD, C), jnp.float32),
        mesh=mesh,
        compiler_params=cp,
        scratch_types=[
            pltpu.VMEM((NPAD,), jnp.float32),
            pltpu.VMEM((K * NPT,), jnp.float32),
            pltpu.VMEM((K * NPT,), jnp.int32),
            pltpu.VMEM((K * NPT,), jnp.int32),
            pltpu.VMEM((K * NPT + LANES,), jnp.float32),
            pltpu.VMEM((K * NPT,), jnp.int32),
            pltpu.VMEM_SHARED((NT * NPAD,), jnp.float32),
            pltpu.VMEM_SHARED((2 * NT * NPAD,), jnp.int32),
            pltpu.VMEM_SHARED((NPAD,), jnp.float32),
            pltpu.SemaphoreType.DMA,
        ],
    )


def kernel(x, edge_index, W, att):
    N, IN = x.shape
    C = W.shape[0]
    E = edge_index.shape[1]

    NPAD = ((N + NT * LANES - 1) // (NT * LANES)) * NT * LANES
    EP = E + N
    EPT = ((EP + NT * 4 * LANES - 1) // (NT * 4 * LANES)) * 4 * LANES
    EPAD = EPT * NT

    WT = W.T
    attd = att[0, 0, :C].reshape(C, 1)
    atts = att[0, 0, C:].reshape(C, 1)

    xp, a1, b1 = _make_tc(N, IN, C)(x, WT, attd, atts)
    a = jnp.pad(a1[:, 0], (0, NPAD - N))
    b = jnp.pad(b1[:, 0], (0, NPAD - N))

    loops = jnp.arange(N, dtype=jnp.int32)
    rowp = jnp.concatenate(
        [edge_index[0], loops,
         jnp.full((EPAD - EP,), N, jnp.int32)])
    colp = jnp.concatenate(
        [edge_index[1], loops,
         jnp.zeros((EPAD - EP,), jnp.int32)])
    packed = rowp | (colp << RBITS)

    out = _make_sc(N, C, EPAD, NPAD)(packed, a, b, xp)
    return out[:N]


# speedup vs baseline: 1.8287x; 1.0381x over previous
import dataclasses
import functools

import jax
import jax.numpy as jnp
from jax import lax
from jax.experimental import pallas as pl
from jax.experimental.pallas import tpu as pltpu
from jax.experimental.pallas import tpu_sc as plsc

K = 5
LANES = 16
NT = 16
MINF = float("-inf")
RBITS = 14


def _tc_body(x_ref, wt_ref, ad_ref, as_ref, xp_ref, a_ref, b_ref):
    xp = jnp.dot(x_ref[...], wt_ref[...], preferred_element_type=jnp.float32)
    xp_ref[...] = xp
    a_ref[...] = jnp.dot(xp, ad_ref[...], preferred_element_type=jnp.float32,
                         precision=jax.lax.Precision.HIGHEST)
    b_ref[...] = jnp.dot(xp, as_ref[...], preferred_element_type=jnp.float32,
                         precision=jax.lax.Precision.HIGHEST)


@functools.lru_cache(maxsize=None)
def _make_tc(N, IN, C):
    BLK = 2000
    assert N % BLK == 0
    grid = (N // BLK,)
    return pl.pallas_call(
        _tc_body,
        grid=grid,
        in_specs=[
            pl.BlockSpec((BLK, IN), lambda i: (i, 0)),
            pl.BlockSpec((IN, C), lambda i: (0, 0)),
            pl.BlockSpec((C, 1), lambda i: (0, 0)),
            pl.BlockSpec((C, 1), lambda i: (0, 0)),
        ],
        out_specs=[
            pl.BlockSpec((BLK, C), lambda i: (i, 0)),
            pl.BlockSpec((BLK, 1), lambda i: (i, 0)),
            pl.BlockSpec((BLK, 1), lambda i: (i, 0)),
        ],
        out_shape=[
            jax.ShapeDtypeStruct((N, C), jnp.float32),
            jax.ShapeDtypeStruct((N, 1), jnp.float32),
            jax.ShapeDtypeStruct((N, 1), jnp.float32),
        ],
    )


@functools.lru_cache(maxsize=None)
def _make_sc(N, C, EPAD, NPAD):
    EPT = EPAD // NT
    NPT = NPAD // NT
    NV_E = EPT // LANES
    NV_N = NPT // LANES
    NV_P = NPAD // LANES
    CH = 16
    NCH = NPT // CH
    CL = C // LANES
    RMASK = (1 << RBITS) - 1

    mesh = plsc.VectorSubcoreMesh(
        core_axis_name="c", subcore_axis_name="s", num_cores=1,
        num_subcores=NT)

    def body(pe_hbm, a_hbm, b_hbm, xp_hbm, out_hbm,
             gm, mk, ck, colk, wk, selcol, sh16f, sh16i, spub, sem):
        tid = lax.axis_index("s")

        def fill(ref, nvec, val, dtype):
            def fb(i, _):
                ref[pl.ds(i * LANES, LANES)] = jnp.full((LANES,), val, dtype)
                return 0
            lax.fori_loop(0, nvec, fb, 0)

        def selection(pev, bnode):
            with jax.named_scope("prologue"):
                pltpu.sync_copy(pe_hbm.at[pl.ds(tid * EPT, EPT)], pev)
                pltpu.sync_copy(b_hbm, bnode)

            def _fix(rows, v, macc):
                def cond(gc):
                    return jnp.any(v > gc)

                def bdy(gc):
                    plsc.store_scatter(macc, [rows], jnp.maximum(gc, v),
                                       mask=v > gc)
                    return plsc.load_gather(macc, [rows])

                lax.while_loop(cond, bdy, plsc.load_gather(macc, [rows]))

            def rmw_max_pair(pairs, macc):
                for rows, v in pairs:
                    g = plsc.load_gather(macc, [rows])
                    plsc.store_scatter(macc, [rows], jnp.maximum(g, v),
                                       mask=v > g)
                lost = None
                for rows, v in pairs:
                    gv = plsc.load_gather(macc, [rows])
                    l = v > gv
                    lost = l if lost is None else (lost | l)

                @pl.when(jnp.any(lost))
                def _slow():
                    for rows, v in pairs:
                        _fix(rows, v, macc)

            def max_pass0(macc):
                def eb(i, _):
                    pairs = []
                    for u in range(8):
                        st = (8 * i + u) * LANES
                        pe = pev[pl.ds(st, LANES)]
                        rows = pe & RMASK
                        cols = lax.shift_right_logical(pe, RBITS)
                        be = plsc.load_gather(bnode, [cols])
                        pairs.append((rows, be))
                    rmw_max_pair(pairs, macc)
                    return 0
                lax.fori_loop(0, NV_E // 8, eb, 0)

            def fused_pass(cnt, colc, macc, with_max):
                ones = jnp.ones((LANES,), jnp.int32)

                def eb(i, _):
                    pairs = []
                    for u in range(8):
                        st = (8 * i + u) * LANES
                        pe = pev[pl.ds(st, LANES)]
                        rows = pe & RMASK
                        cols = lax.shift_right_logical(pe, RBITS)
                        be = plsc.load_gather(bnode, [cols])
                        mv = plsc.load_gather(gm, [rows])
                        eq = be == mv
                        plsc.addupdate_scatter(cnt, [rows], ones, mask=eq)
                        plsc.store_scatter(colc, [rows], cols, mask=eq)
                        if with_max:
                            v = jnp.where(
                                be < mv, be,
                                jnp.full((LANES,), MINF, jnp.float32))
                            pairs.append((rows, v))
                    if with_max:
                        rmw_max_pair(pairs, macc)
                    return 0
                lax.fori_loop(0, NV_E // 8, eb, 0)

            def merge_max(k, macc):
                stagef = macc
                pltpu.sync_copy(macc, sh16f.at[pl.ds(tid * NPAD, NPAD)])
                plsc.subcore_barrier()
                for j in range(NT):
                    pltpu.sync_copy(
                        sh16f.at[pl.ds(j * NPAD + tid * NPT, NPT)],
                        stagef.at[pl.ds(j * NPT, NPT)])

                def rb(v, _):
                    st = v * LANES
                    red = stagef[pl.ds(st, LANES)]
                    for j in range(1, NT):
                        red = jnp.maximum(red,
                                          stagef[pl.ds(j * NPT + st, LANES)])
                    mk[pl.ds(k * NPT + st, LANES)] = red
                    return 0
                lax.fori_loop(0, NV_N, rb, 0)
                plsc.subcore_barrier()
                pltpu.sync_copy(mk.at[pl.ds(k * NPT, NPT)],
                                spub.at[pl.ds(tid * NPT, NPT)])
                plsc.subcore_barrier()
                pltpu.sync_copy(spub, gm)
                plsc.subcore_barrier()

            def merge_cnts(k, cnt, colc):
                stagei = cnt
                pltpu.sync_copy(cnt, sh16i.at[pl.ds(tid * 2 * NPAD, NPAD)])
                pltpu.sync_copy(colc,
                                sh16i.at[pl.ds(tid * 2 * NPAD + NPAD, NPAD)])
                plsc.subcore_barrier()
                for j in range(NT):
                    pltpu.sync_copy(
                        sh16i.at[pl.ds(2 * j * NPAD + tid * NPT, NPT)],
                        stagei.at[pl.ds(j * NPT, NPT)])

                def rbc(v, _):
                    st = v * LANES
                    redc = stagei[pl.ds(st, LANES)]
                    for j in range(1, NT):
                        redc = redc + stagei[pl.ds(j * NPT + st, LANES)]
                    ck[pl.ds(k * NPT + st, LANES)] = redc
                    return 0
                lax.fori_loop(0, NV_N, rbc, 0)
                for j in range(NT):
                    pltpu.sync_copy(
                        sh16i.at[pl.ds((2 * j + 1) * NPAD + tid * NPT, NPT)],
                        stagei.at[pl.ds(j * NPT, NPT)])

                def rbl(v, _):
                    st = v * LANES
                    redl = stagei[pl.ds(st, LANES)]
                    for j in range(1, NT):
                        redl = jnp.maximum(redl,
                                           stagei[pl.ds(j * NPT + st, LANES)])
                    colk[pl.ds(k * NPT + st, LANES)] = redl
                    return 0
                lax.fori_loop(0, NV_N, rbl, 0)
                plsc.subcore_barrier()

            def rounds(cnt, colc, macc):
                with jax.named_scope("p0_fill"):
                    fill(macc, NV_P, MINF, jnp.float32)
                with jax.named_scope("p0_max"):
                    max_pass0(macc)
                with jax.named_scope("p0_merge"):
                    merge_max(0, macc)
                for k in range(1, K + 1):
                    with_max = k < K
                    with jax.named_scope(f"r{k}_fill"):
                        def fb(i, _, _wm=with_max):
                            sl = pl.ds(i * LANES, LANES)
                            cnt[sl] = jnp.zeros((LANES,), jnp.int32)
                            colc[sl] = jnp.full((LANES,), -1, jnp.int32)
                            if _wm:
                                macc[sl] = jnp.full((LANES,), MINF,
                                                    jnp.float32)
                            return 0
                        lax.fori_loop(0, NV_P, fb, 0)
                    with jax.named_scope(f"r{k}_pass"):
                        fused_pass(cnt, colc, macc, with_max)
                    with jax.named_scope(f"r{k}_mcnt"):
                        merge_cnts(k - 1, cnt, colc)
                    if with_max:
                        with jax.named_scope(f"r{k}_mmax"):
                            merge_max(k, macc)
            pl.run_scoped(rounds,
                          pltpu.VMEM((NPAD,), jnp.int32),
                          pltpu.VMEM((NPAD,), jnp.int32),
                          pltpu.VMEM((NPAD,), jnp.float32))

        pl.run_scoped(selection,
                      pltpu.VMEM((EPT,), jnp.int32),
                      pltpu.VMEM((NPAD,), jnp.float32))

        def weights(anode):
            pltpu.sync_copy(a_hbm.at[pl.ds(tid * NPT, NPT)], anode)

            def wb(v, _):
                st = v * LANES
                a16 = anode[pl.ds(st, LANES)]
                rem = jnp.full((LANES,), K, jnp.int32)
                denom = jnp.zeros((LANES,), jnp.float32)
                exs = []
                s0 = None
                for k in range(K):
                    m = mk[pl.ds(k * NPT + st, LANES)]
                    c = ck[pl.ds(k * NPT + st, LANES)]
                    s = a16 + m
                    s = jnp.where(s >= 0, s, 0.2 * s)
                    if k == 0:
                        s0 = s
                    t = jnp.minimum(c, jnp.maximum(rem, 0))
                    rem = rem - t
                    valid = t > 0
                    ex = jnp.where(valid,
                                   t.astype(jnp.float32) * jnp.exp(s - s0),
                                   jnp.zeros((LANES,), jnp.float32))
                    denom = denom + ex
                    exs.append(ex)
                    cv = colk[pl.ds(k * NPT + st, LANES)]
                    selcol[pl.ds(k * NPT + st, LANES)] = jnp.where(
                        cv >= 0, cv, jnp.zeros((LANES,), jnp.int32))
                for k in range(K):
                    wk[pl.ds(k * NPT + st, LANES)] = exs[k] / denom
                return 0
            lax.fori_loop(0, NV_N, wb, 0)
        with jax.named_scope("weights"):
            pl.run_scoped(weights, pltpu.VMEM((NPT,), jnp.float32))

        def aggregate(rowbuf, outchunk):
            def fire(ch, buf):
                descs = []
                for k in range(K):
                    d = pltpu.async_copy(
                        xp_hbm.at[selcol.at[pl.ds(k * NPT + ch * CH, CH)]],
                        rowbuf.at[pl.ds((buf * K + k) * CH, CH)], sem)
                    descs.append(d)
                return descs

            pending = fire(0, 0)
            for ch in range(NCH):
                buf = ch % 2
                for d in pending:
                    d.wait()
                if ch + 1 < NCH:
                    pending = fire(ch + 1, 1 - buf)

                def nbody(n, _):
                    zero = jnp.zeros((LANES,), jnp.float32)
                    accs0 = tuple(zero for _ in range(CL))

                    def kbody(k, accs):
                        wv = wk[pl.ds(k * NPT + ch * CH + n, LANES)]
                        wb16 = jnp.full((LANES,), wv[0])
                        out = []
                        for c in range(CL):
                            r = rowbuf[(buf * K + k) * CH + n,
                                       pl.ds(c * LANES, LANES)]
                            out.append(accs[c] + wb16 * r)
                        return tuple(out)

                    accs = lax.fori_loop(0, K, kbody, accs0)
                    for c in range(CL):
                        v = accs[c]
                        ev = jnp.where(
                            v > 0, v,
                            jnp.exp(jnp.minimum(v, 0.0)) - 1.0)
                        outchunk[n, pl.ds(c * LANES, LANES)] = ev
                    return 0
                lax.fori_loop(0, CH, nbody, 0)
                pltpu.sync_copy(outchunk,
                                out_hbm.at[pl.ds(tid * NPT + ch * CH, CH), :])
        with jax.named_scope("agg"):
            pl.run_scoped(aggregate,
                          pltpu.VMEM((2 * K * CH, C), jnp.float32),
                          pltpu.VMEM((CH, C), jnp.float32))

    cp = pltpu.CompilerParams()
    if "needs_layout_passes" in pltpu.CompilerParams.__dataclass_fields__:
        cp = dataclasses.replace(cp, needs_layout_passes=False)

    return pl.kernel(
        body,
        out_type=jax.ShapeDtypeStruct((NPAD, C), jnp.float32),
        mesh=mesh,
        compiler_params=cp,
        scratch_types=[
            pltpu.VMEM((NPAD,), jnp.float32),
            pltpu.VMEM((K * NPT,), jnp.float32),
            pltpu.VMEM((K * NPT,), jnp.int32),
            pltpu.VMEM((K * NPT,), jnp.int32),
            pltpu.VMEM((K * NPT + LANES,), jnp.float32),
            pltpu.VMEM((K * NPT,), jnp.int32),
            pltpu.VMEM_SHARED((NT * NPAD,), jnp.float32),
            pltpu.VMEM_SHARED((2 * NT * NPAD,), jnp.int32),
            pltpu.VMEM_SHARED((NPAD,), jnp.float32),
            pltpu.SemaphoreType.DMA,
        ],
    )


def kernel(x, edge_index, W, att):
    N, IN = x.shape
    C = W.shape[0]
    E = edge_index.shape[1]

    NPAD = ((N + NT * LANES - 1) // (NT * LANES)) * NT * LANES
    EP = E + N
    EPT = ((EP + NT * 8 * LANES - 1) // (NT * 8 * LANES)) * 8 * LANES
    EPAD = EPT * NT

    WT = W.T
    attd = att[0, 0, :C].reshape(C, 1)
    atts = att[0, 0, C:].reshape(C, 1)

    xp, a1, b1 = _make_tc(N, IN, C)(x, WT, attd, atts)
    a = jnp.pad(a1[:, 0], (0, NPAD - N))
    b = jnp.pad(b1[:, 0], (0, NPAD - N))

    loops = jnp.arange(N, dtype=jnp.int32)
    rowp = jnp.concatenate(
        [edge_index[0], loops,
         jnp.full((EPAD - EP,), N, jnp.int32)])
    colp = jnp.concatenate(
        [edge_index[1], loops,
         jnp.zeros((EPAD - EP,), jnp.int32)])
    packed = rowp | (colp << RBITS)

    out = _make_sc(N, C, EPAD, NPAD)(packed, a, b, xp)
    return out[:N]
